# baseline (speedup 1.0000x reference)
import functools

import jax
import jax.numpy as jnp
from jax import lax
from jax.experimental import pallas as pl
from jax.experimental.pallas import tpu as pltpu
from jax.experimental.pallas import tpu_sc as plsc

_NUM_EXPERTS = 64
_TOP_K = 8
_BLOCK_T = 2048
_N_TOK = 8192
_NW = 32
_TPW = _N_TOK // _NW


def _probs_body(h_ref, w_ref, probs_ref, probs_t_ref):
    h = h_ref[...]
    w = w_ref[...]
    logits_t = jax.lax.dot_general(
        w, h, (((1,), (1,)), ((), ())), preferred_element_type=jnp.float32
    )
    m = jnp.max(logits_t, axis=0, keepdims=True)
    e = jnp.exp(logits_t - m)
    s = jnp.sum(e, axis=0, keepdims=True)
    probs_t = e / s
    probs_ref[...] = probs_t.T
    probs_t_ref[...] = probs_t


def _tree_reduce(vals, op):
    while len(vals) > 1:
        nxt = [op(vals[i], vals[i + 1]) for i in range(0, len(vals) - 1, 2)]
        if len(vals) % 2:
            nxt.append(vals[-1])
        vals = nxt
    return vals[0]


@functools.partial(
    pl.kernel,
    mesh=plsc.VectorSubcoreMesh(core_axis_name="c", subcore_axis_name="s"),
    out_type=[
        jax.ShapeDtypeStruct((_TOP_K, _N_TOK), jnp.float32),
        jax.ShapeDtypeStruct((_TOP_K, _N_TOK), jnp.int32),
    ],
    scratch_types=[
        pltpu.VMEM((_NUM_EXPERTS, _TPW), jnp.float32),
        pltpu.VMEM((_TOP_K, _TPW), jnp.float32),
        pltpu.VMEM((_TOP_K, _TPW), jnp.int32),
    ],
)
def _sc_topk(probs_t_hbm, tkw_hbm, tki_hbm, probs_v, tkw_v, tki_v):
    wid = lax.axis_index("s") * 2 + lax.axis_index("c")
    base = wid * _TPW
    pltpu.sync_copy(probs_t_hbm.at[:, pl.ds(base, _TPW)], probs_v)

    def group_body(g, carry):
        t0 = g * 16
        work = [probs_v[e, pl.ds(t0, 16)] for e in range(_NUM_EXPERTS)]
        for r in range(_TOP_K):
            cur = _tree_reduce(list(work), jnp.maximum)
            cand = [
                jnp.where(
                    work[e] == cur,
                    jnp.full((16,), e, jnp.int32),
                    jnp.full((16,), _NUM_EXPERTS, jnp.int32),
                )
                for e in range(_NUM_EXPERTS)
            ]
            idx = _tree_reduce(cand, jnp.minimum)
            tkw_v[r, pl.ds(t0, 16)] = cur
            tki_v[r, pl.ds(t0, 16)] = idx
            if r != _TOP_K - 1:
                work = [
                    jnp.where(idx == e, jnp.float32(-1.0), work[e])
                    for e in range(_NUM_EXPERTS)
                ]
        return carry

    lax.fori_loop(0, _TPW // 16, group_body, 0)
    pltpu.sync_copy(tkw_v, tkw_hbm.at[:, pl.ds(base, _TPW)])
    pltpu.sync_copy(tki_v, tki_hbm.at[:, pl.ds(base, _TPW)])


def kernel(hidden_states, W_router):
    d_model = hidden_states.shape[-1]
    h2d = hidden_states.reshape(-1, d_model)
    n_tok = h2d.shape[0]
    grid = (n_tok // _BLOCK_T,)
    probs, probs_t = pl.pallas_call(
        _probs_body,
        grid=grid,
        in_specs=[
            pl.BlockSpec((_BLOCK_T, d_model), lambda i: (i, 0)),
            pl.BlockSpec((_NUM_EXPERTS, d_model), lambda i: (0, 0)),
        ],
        out_specs=[
            pl.BlockSpec((_BLOCK_T, _NUM_EXPERTS), lambda i: (i, 0)),
            pl.BlockSpec((_NUM_EXPERTS, _BLOCK_T), lambda i: (0, i)),
        ],
        out_shape=[
            jax.ShapeDtypeStruct((n_tok, _NUM_EXPERTS), jnp.float32),
            jax.ShapeDtypeStruct((_NUM_EXPERTS, n_tok), jnp.float32),
        ],
    )(h2d, W_router)
    tkw_t, tki_t = _sc_topk(probs_t)
    return (probs, tkw_t.T, tki_t.T)


# speedup vs baseline: 1.2615x; 1.2615x over previous
import jax
import jax.numpy as jnp
from jax.experimental import pallas as pl
from jax.experimental.pallas import tpu as pltpu

_NUM_EXPERTS = 64
_TOP_K = 8
_BLOCK_T = 2048
_BLOCK_K = 1024


def _gate_body(h_ref, w_ref, probs_ref, tkw_ref, tki_ref, acc_ref):
    k = pl.program_id(1)
    nk = pl.num_programs(1)
    partial = jax.lax.dot_general(
        w_ref[...], h_ref[...], (((1,), (1,)), ((), ())),
        preferred_element_type=jnp.float32,
    )

    @pl.when(k == 0)
    def _():
        acc_ref[...] = partial

    @pl.when(k != 0)
    def _():
        acc_ref[...] += partial

    @pl.when(k == nk - 1)
    def _():
        logits_t = acc_ref[...]
        m = jnp.max(logits_t, axis=0, keepdims=True)
        e = jnp.exp(logits_t - m)
        s = jnp.sum(e, axis=0, keepdims=True)
        probs_t = e / s
        probs_ref[...] = probs_t.T

        iota = jax.lax.broadcasted_iota(jnp.int32, probs_t.shape, 0)
        work = probs_t
        w_rows = []
        i_rows = []
        for _ in range(_TOP_K):
            cur = jnp.max(work, axis=0, keepdims=True)
            idx = jnp.min(
                jnp.where(work == cur, iota, _NUM_EXPERTS), axis=0, keepdims=True
            )
            w_rows.append(cur)
            i_rows.append(idx)
            work = jnp.where(iota == idx, -1.0, work)
        tkw_ref[...] = jnp.concatenate(w_rows, axis=0).T
        tki_ref[...] = jnp.concatenate(i_rows, axis=0).T


def kernel(hidden_states, W_router):
    d_model = hidden_states.shape[-1]
    h2d = hidden_states.reshape(-1, d_model)
    n_tok = h2d.shape[0]
    grid = (n_tok // _BLOCK_T, d_model // _BLOCK_K)
    probs, tkw, tki = pl.pallas_call(
        _gate_body,
        grid=grid,
        in_specs=[
            pl.BlockSpec((_BLOCK_T, _BLOCK_K), lambda i, j: (i, j)),
            pl.BlockSpec((_NUM_EXPERTS, _BLOCK_K), lambda i, j: (0, j)),
        ],
        out_specs=[
            pl.BlockSpec((_BLOCK_T, _NUM_EXPERTS), lambda i, j: (i, 0)),
            pl.BlockSpec((_BLOCK_T, _TOP_K), lambda i, j: (i, 0)),
            pl.BlockSpec((_BLOCK_T, _TOP_K), lambda i, j: (i, 0)),
        ],
        out_shape=[
            jax.ShapeDtypeStruct((n_tok, _NUM_EXPERTS), jnp.float32),
            jax.ShapeDtypeStruct((n_tok, _TOP_K), jnp.float32),
            jax.ShapeDtypeStruct((n_tok, _TOP_K), jnp.int32),
        ],
        scratch_shapes=[pltpu.VMEM((_NUM_EXPERTS, _BLOCK_T), jnp.float32)],
    )(h2d, W_router)
    return (probs, tkw, tki)


# speedup vs baseline: 1.4245x; 1.1292x over previous
import jax
import jax.numpy as jnp
from jax.experimental import pallas as pl

_NUM_EXPERTS = 64
_TOP_K = 8
_BLOCK_T = 2048


def _gate_body(h_ref, w_ref, probs_ref, tkw_ref, tki_ref):
    h = h_ref[...]
    w = w_ref[...]
    logits_t = jax.lax.dot_general(
        w, h, (((1,), (1,)), ((), ())), preferred_element_type=jnp.float32
    )
    m = jnp.max(logits_t, axis=0, keepdims=True)
    e = jnp.exp(logits_t - m)
    s = jnp.sum(e, axis=0, keepdims=True)
    probs_t = e / s
    probs_ref[...] = probs_t.T

    iota = jax.lax.broadcasted_iota(jnp.int32, probs_t.shape, 0)
    work = probs_t
    w_rows = []
    i_rows = []
    for _ in range(_TOP_K):
        cur = jnp.max(work, axis=0, keepdims=True)
        idx = jnp.min(
            jnp.where(work == cur, iota, _NUM_EXPERTS), axis=0, keepdims=True
        )
        w_rows.append(cur)
        i_rows.append(idx)
        work = jnp.where(iota == idx, -1.0, work)
    tkw_ref[...] = jnp.concatenate(w_rows, axis=0).T
    tki_ref[...] = jnp.concatenate(i_rows, axis=0).T


def kernel(hidden_states, W_router):
    d_model = hidden_states.shape[-1]
    h2d = hidden_states.reshape(-1, d_model)
    n_tok = h2d.shape[0]
    grid = (n_tok // _BLOCK_T,)
    probs, tkw, tki = pl.pallas_call(
        _gate_body,
        grid=grid,
        in_specs=[
            pl.BlockSpec((_BLOCK_T, d_model), lambda i: (i, 0)),
            pl.BlockSpec((_NUM_EXPERTS, d_model), lambda i: (0, 0)),
        ],
        out_specs=[
            pl.BlockSpec((_BLOCK_T, _NUM_EXPERTS), lambda i: (i, 0)),
            pl.BlockSpec((_BLOCK_T, _TOP_K), lambda i: (i, 0)),
            pl.BlockSpec((_BLOCK_T, _TOP_K), lambda i: (i, 0)),
        ],
        out_shape=[
            jax.ShapeDtypeStruct((n_tok, _NUM_EXPERTS), jnp.float32),
            jax.ShapeDtypeStruct((n_tok, _TOP_K), jnp.float32),
            jax.ShapeDtypeStruct((n_tok, _TOP_K), jnp.int32),
        ],
    )(h2d, W_router)
    return (probs, tkw, tki)
